# baseline (speedup 1.0000x reference)
import jax
import jax.numpy as jnp
from jax import lax
from jax.experimental import pallas as pl
from jax.experimental.pallas import tpu as pltpu
from jax.experimental.pallas import tpu_sc as plsc

VOCAB = 128
DIM = 32
IW = 1024


def kernel(x, table):
    nrows, seq = x.shape
    xt = jnp.transpose(x.astype(jnp.int32))
    tab_flat = jnp.transpose(table.astype(jnp.float32)).reshape(VOCAB * DIM)

    mesh = plsc.VectorSubcoreMesh(core_axis_name="core",
                                  subcore_axis_name="subcore")

    out5 = jax.ShapeDtypeStruct((seq, DIM // 8, nrows // VOCAB, 8, VOCAB),
                                jnp.float32)

    @pl.kernel(out_type=out5,
               mesh=mesh,
               compiler_params=pltpu.CompilerParams(
                   use_tc_tiling_on_sc=False, needs_layout_passes=False),
               scratch_types=[pltpu.VMEM((VOCAB * DIM,), jnp.float32)])
    def gather_kernel(table_hbm, i_hbm, o_hbm, tab_v):
        pltpu.sync_copy(table_hbm, tab_v)

        def body(i_vmem, o_vmem):
            @plsc.parallel_loop(0, IW, step=16, unroll=32)
            def _(i0):
                vidx = i_vmem[0, pl.ds(i0, 16)]
                b = i0 // VOCAB
                l0 = i0 % VOCAB
                for c in range(DIM):
                    o_vmem[0, c // 8, b, c % 8, pl.ds(l0, 16)] = (
                        plsc.load_gather(tab_v, [vidx + c * VOCAB]))

        pltpu.emit_pipeline(
            body,
            grid=(seq, nrows // IW),
            in_specs=[pl.BlockSpec((1, IW), lambda j, i: (j, i))],
            out_specs=[pl.BlockSpec((1, DIM // 8, IW // VOCAB, 8, VOCAB),
                                    lambda j, i: (j, 0, i, 0, 0))],
            core_axis_name=("core", "subcore"),
            dimension_semantics=(pltpu.PARALLEL, pltpu.PARALLEL),
        )(i_hbm, o_hbm)

    out_t = gather_kernel(tab_flat, xt)
    out = jnp.transpose(out_t, (2, 4, 0, 1, 3)).reshape(nrows, seq, DIM)
    return out


# speedup vs baseline: 1.6329x; 1.6329x over previous
import jax
import jax.numpy as jnp
from jax import lax
from jax.experimental import pallas as pl
from jax.experimental.pallas import tpu as pltpu
from jax.experimental.pallas import tpu_sc as plsc

VOCAB = 128
DIM = 32
IW = 512


def kernel(x, table):
    nrows, seq = x.shape
    xt = jnp.transpose(x.astype(jnp.int32))
    tab_flat = jnp.transpose(table.astype(jnp.float32)).reshape(VOCAB * DIM)

    mesh = plsc.VectorSubcoreMesh(core_axis_name="core",
                                  subcore_axis_name="subcore")

    out5 = jax.ShapeDtypeStruct((seq, DIM // 8, nrows // VOCAB, 8, VOCAB),
                                jnp.float32)

    @pl.kernel(out_type=out5,
               mesh=mesh,
               compiler_params=pltpu.CompilerParams(
                   use_tc_tiling_on_sc=False, needs_layout_passes=False),
               scratch_types=[pltpu.VMEM((VOCAB * DIM,), jnp.float32)])
    def gather_kernel(table_hbm, i_hbm, o_hbm, tab_v):
        pltpu.sync_copy(table_hbm, tab_v)

        def body(i_vmem, o_vmem):
            @plsc.parallel_loop(0, IW, step=16, unroll=16)
            def _(i0):
                vidx = i_vmem[0, pl.ds(i0, 16)]
                b = i0 // VOCAB
                l0 = i0 % VOCAB
                for c in range(DIM):
                    o_vmem[0, c // 8, b, c % 8, pl.ds(l0, 16)] = (
                        plsc.load_gather(tab_v, [vidx + c * VOCAB]))

        pltpu.emit_pipeline(
            body,
            grid=(seq, nrows // IW),
            in_specs=[pl.BlockSpec((1, IW), lambda j, i: (j, i))],
            out_specs=[pl.BlockSpec((1, DIM // 8, IW // VOCAB, 8, VOCAB),
                                    lambda j, i: (j, 0, i, 0, 0))],
            core_axis_name=("core", "subcore"),
            dimension_semantics=(pltpu.PARALLEL, pltpu.PARALLEL),
        )(i_hbm, o_hbm)

    out_t = gather_kernel(tab_flat, xt)
    out = jnp.transpose(out_t, (2, 4, 0, 1, 3)).reshape(nrows, seq, DIM)
    return out
